# baseline (device time: 152634 ns/iter reference)
import jax
import jax.numpy as jnp
from jax import lax
from jax.experimental import pallas as pl
from jax.experimental.pallas import tpu as pltpu

NC = 16


def kernel(A, B):
    m, k = A.shape
    k2, n = B.shape
    assert k == k2
    assert m % NC == 0
    mc = m // NC

    def body(a_ref, b_ref, out_ref, own_buf, send_buf, recv_buf, out_stage,
             sscale_buf, rscale_buf, bbf_buf, store_sems,
             dsend_sems, drecv_sems, ssend_sems, srecv_sems):
        my_x = lax.axis_index("x")
        my_y = lax.axis_index("y")
        nbr = (my_x, 1 - my_y)

        barrier_sem = pltpu.get_barrier_semaphore()
        pl.semaphore_signal(
            barrier_sem, inc=1, device_id=nbr,
            device_id_type=pl.DeviceIdType.MESH,
        )
        pl.semaphore_wait(barrier_sem, 1)

        bbf_buf[...] = b_ref[...].astype(jnp.bfloat16)

        def dot_chunk(c):
            sl = pl.ds(c * mc, mc)
            p = jnp.dot(
                a_ref[sl, :].astype(jnp.bfloat16), bbf_buf[...],
                preferred_element_type=jnp.float32,
            )
            own_buf[c % 2] = p
            amax = jnp.max(jnp.abs(p)) + 1e-30
            send_buf[c % 2] = jnp.round(p * (127.0 / amax)).astype(jnp.int8)
            sscale_buf[c % 2] = jnp.full((8, 128), amax / 127.0, jnp.float32)

        def make_rdmas(c):
            data = pltpu.make_async_remote_copy(
                src_ref=send_buf.at[c % 2],
                dst_ref=recv_buf.at[c % 4],
                send_sem=dsend_sems.at[c % 2],
                recv_sem=drecv_sems.at[c % 4],
                device_id=nbr,
                device_id_type=pl.DeviceIdType.MESH,
            )
            scale = pltpu.make_async_remote_copy(
                src_ref=sscale_buf.at[c % 2],
                dst_ref=rscale_buf.at[c % 4],
                send_sem=ssend_sems.at[c % 2],
                recv_sem=srecv_sems.at[c % 4],
                device_id=nbr,
                device_id_type=pl.DeviceIdType.MESH,
            )
            return (data, scale)

        rdmas = []
        stores = []

        def start(c):
            rdmas.append(make_rdmas(c))
            rdmas[c][0].start()
            rdmas[c][1].start()

        dot_chunk(0)
        start(0)

        for c in range(NC):
            if c + 1 < NC:
                if c >= 1:
                    rdmas[c - 1][0].wait_send()
                    rdmas[c - 1][1].wait_send()
                dot_chunk(c + 1)
                start(c + 1)
            rdmas[c][0].wait_recv()
            rdmas[c][1].wait_recv()
            if c >= 2:
                stores[c - 2].wait()
            out_stage[c % 2] = (
                own_buf[c % 2]
                + recv_buf[c % 4].astype(jnp.float32) * rscale_buf[c % 4, 0:1, 0:1]
            )
            store = pltpu.make_async_copy(
                out_stage.at[c % 2],
                out_ref.at[pl.ds(c * mc, mc), :],
                store_sems.at[c % 2],
            )
            stores.append(store)
            store.start()

        for c in (NC - 2, NC - 1):
            rdmas[c][0].wait_send()
            rdmas[c][1].wait_send()
            stores[c].wait()

    return pl.pallas_call(
        body,
        out_shape=jax.ShapeDtypeStruct((m, n), jnp.float32),
        in_specs=[
            pl.BlockSpec(memory_space=pltpu.VMEM),
            pl.BlockSpec(memory_space=pltpu.VMEM),
        ],
        out_specs=pl.BlockSpec(memory_space=pltpu.MemorySpace.HBM),
        scratch_shapes=[
            pltpu.VMEM((2, mc, n), jnp.float32),
            pltpu.VMEM((2, mc, n), jnp.int8),
            pltpu.VMEM((4, mc, n), jnp.int8),
            pltpu.VMEM((2, mc, n), jnp.float32),
            pltpu.VMEM((2, 8, 128), jnp.float32),
            pltpu.VMEM((4, 8, 128), jnp.float32),
            pltpu.VMEM((k, n), jnp.bfloat16),
            pltpu.SemaphoreType.DMA((2,)),
            pltpu.SemaphoreType.DMA((2,)),
            pltpu.SemaphoreType.DMA((4,)),
            pltpu.SemaphoreType.DMA((2,)),
            pltpu.SemaphoreType.DMA((4,)),
        ],
        compiler_params=pltpu.CompilerParams(
            collective_id=0, vmem_limit_bytes=62 * 1024 * 1024
        ),
    )(A, B)


# device time: 151182 ns/iter; 1.0096x vs baseline; 1.0096x over previous
import jax
import jax.numpy as jnp
from jax import lax
from jax.experimental import pallas as pl
from jax.experimental.pallas import tpu as pltpu

NC = 8


def kernel(A, B):
    m, k = A.shape
    k2, n = B.shape
    assert k == k2
    assert m % NC == 0
    mc = m // NC

    def body(a_ref, b_ref, out_ref, own_buf, send_buf, recv_buf,
             sscale_buf, rscale_buf, store_sems,
             dsend_sems, drecv_sems, ssend_sems, srecv_sems):
        my_x = lax.axis_index("x")
        my_y = lax.axis_index("y")
        nbr = (my_x, 1 - my_y)

        barrier_sem = pltpu.get_barrier_semaphore()
        pl.semaphore_signal(
            barrier_sem, inc=1, device_id=nbr,
            device_id_type=pl.DeviceIdType.MESH,
        )
        pl.semaphore_wait(barrier_sem, 1)

        def dot_chunk(c):
            sl = pl.ds(c * mc, mc)
            p = jnp.dot(
                a_ref[sl, :], b_ref[:, :], preferred_element_type=jnp.float32
            )
            own_buf[c % 2] = p
            amax = jnp.max(jnp.abs(p)) + 1e-30
            send_buf[c % 2] = jnp.round(p * (127.0 / amax)).astype(jnp.int8)
            sscale_buf[c % 2] = jnp.full((8, 128), amax / 127.0, jnp.float32)

        def make_rdmas(c):
            data = pltpu.make_async_remote_copy(
                src_ref=send_buf.at[c % 2],
                dst_ref=recv_buf.at[c % 4],
                send_sem=dsend_sems.at[c % 2],
                recv_sem=drecv_sems.at[c % 4],
                device_id=nbr,
                device_id_type=pl.DeviceIdType.MESH,
            )
            scale = pltpu.make_async_remote_copy(
                src_ref=sscale_buf.at[c % 2],
                dst_ref=rscale_buf.at[c % 4],
                send_sem=ssend_sems.at[c % 2],
                recv_sem=srecv_sems.at[c % 4],
                device_id=nbr,
                device_id_type=pl.DeviceIdType.MESH,
            )
            return (data, scale)

        rdmas = []
        stores = []

        def start(c):
            rdmas.append(make_rdmas(c))
            rdmas[c][0].start()
            rdmas[c][1].start()

        dot_chunk(0)
        start(0)

        for c in range(NC):
            if c + 1 < NC:
                if c >= 1:
                    rdmas[c - 1][0].wait_send()
                    rdmas[c - 1][1].wait_send()
                    stores[c - 1].wait()
                dot_chunk(c + 1)
                start(c + 1)
            rdmas[c][0].wait_recv()
            rdmas[c][1].wait_recv()
            own_buf[c % 2] = (
                own_buf[c % 2]
                + recv_buf[c % 4].astype(jnp.float32) * rscale_buf[c % 4, 0:1, 0:1]
            )
            store = pltpu.make_async_copy(
                own_buf.at[c % 2],
                out_ref.at[pl.ds(c * mc, mc), :],
                store_sems.at[c % 2],
            )
            stores.append(store)
            store.start()

        for c in (NC - 2, NC - 1):
            rdmas[c][0].wait_send()
            rdmas[c][1].wait_send()
            stores[c].wait()

    return pl.pallas_call(
        body,
        out_shape=jax.ShapeDtypeStruct((m, n), jnp.float32),
        in_specs=[
            pl.BlockSpec(memory_space=pltpu.VMEM),
            pl.BlockSpec(memory_space=pltpu.VMEM),
        ],
        out_specs=pl.BlockSpec(memory_space=pltpu.MemorySpace.HBM),
        scratch_shapes=[
            pltpu.VMEM((2, mc, n), jnp.float32),
            pltpu.VMEM((2, mc, n), jnp.int8),
            pltpu.VMEM((4, mc, n), jnp.int8),
            pltpu.VMEM((2, 8, 128), jnp.float32),
            pltpu.VMEM((4, 8, 128), jnp.float32),
            pltpu.SemaphoreType.DMA((2,)),
            pltpu.SemaphoreType.DMA((2,)),
            pltpu.SemaphoreType.DMA((4,)),
            pltpu.SemaphoreType.DMA((2,)),
            pltpu.SemaphoreType.DMA((4,)),
        ],
        compiler_params=pltpu.CompilerParams(
            collective_id=0, vmem_limit_bytes=62 * 1024 * 1024
        ),
    )(A, B)


# device time: 147630 ns/iter; 1.0339x vs baseline; 1.0241x over previous
import jax
import jax.numpy as jnp
from jax import lax
from jax.experimental import pallas as pl
from jax.experimental.pallas import tpu as pltpu

CHUNKS = (128, 256, 448, 480, 480, 480, 448, 224, 128)
NC = len(CHUNKS)


def kernel(A, B):
    m, k = A.shape
    k2, n = B.shape
    assert k == k2
    assert sum(CHUNKS) == m
    mc = max(CHUNKS)
    offs = [sum(CHUNKS[:i]) for i in range(NC)]

    def body(a_ref, b_ref, out_ref, own_buf, send_buf, recv_buf,
             sscale_buf, rscale_buf, store_sems,
             dsend_sems, drecv_sems, ssend_sems, srecv_sems):
        my_x = lax.axis_index("x")
        my_y = lax.axis_index("y")
        nbr = (my_x, 1 - my_y)

        barrier_sem = pltpu.get_barrier_semaphore()
        pl.semaphore_signal(
            barrier_sem, inc=1, device_id=nbr,
            device_id_type=pl.DeviceIdType.MESH,
        )
        pl.semaphore_wait(barrier_sem, 1)

        def dot_chunk(c):
            sz = CHUNKS[c]
            p = jnp.dot(
                a_ref[pl.ds(offs[c], sz), :], b_ref[:, :],
                preferred_element_type=jnp.float32,
            )
            own_buf[c % 2, pl.ds(0, sz), :] = p
            amax = jnp.max(jnp.abs(p)) + 1e-30
            send_buf[c % 2, pl.ds(0, sz), :] = jnp.round(
                p * (127.0 / amax)
            ).astype(jnp.int8)
            sscale_buf[c % 2] = jnp.full((8, 128), amax / 127.0, jnp.float32)

        def make_rdmas(c):
            sz = CHUNKS[c]
            data = pltpu.make_async_remote_copy(
                src_ref=send_buf.at[c % 2, pl.ds(0, sz), :],
                dst_ref=recv_buf.at[c % 4, pl.ds(0, sz), :],
                send_sem=dsend_sems.at[c % 2],
                recv_sem=drecv_sems.at[c % 4],
                device_id=nbr,
                device_id_type=pl.DeviceIdType.MESH,
            )
            scale = pltpu.make_async_remote_copy(
                src_ref=sscale_buf.at[c % 2],
                dst_ref=rscale_buf.at[c % 4],
                send_sem=ssend_sems.at[c % 2],
                recv_sem=srecv_sems.at[c % 4],
                device_id=nbr,
                device_id_type=pl.DeviceIdType.MESH,
            )
            return (data, scale)

        rdmas = []
        stores = []

        def start(c):
            rdmas.append(make_rdmas(c))
            rdmas[c][0].start()
            rdmas[c][1].start()

        dot_chunk(0)
        start(0)

        for c in range(NC):
            if c + 1 < NC:
                if c >= 1:
                    rdmas[c - 1][0].wait_send()
                    rdmas[c - 1][1].wait_send()
                    stores[c - 1].wait()
                dot_chunk(c + 1)
                start(c + 1)
            rdmas[c][0].wait_recv()
            rdmas[c][1].wait_recv()
            sz = CHUNKS[c]
            own_buf[c % 2, pl.ds(0, sz), :] = (
                own_buf[c % 2, pl.ds(0, sz), :]
                + recv_buf[c % 4, pl.ds(0, sz), :].astype(jnp.float32)
                * rscale_buf[c % 4, 0:1, 0:1]
            )
            store = pltpu.make_async_copy(
                own_buf.at[c % 2, pl.ds(0, sz), :],
                out_ref.at[pl.ds(offs[c], sz), :],
                store_sems.at[c % 2],
            )
            stores.append(store)
            store.start()

        for c in (NC - 2, NC - 1):
            rdmas[c][0].wait_send()
            rdmas[c][1].wait_send()
            stores[c].wait()

    return pl.pallas_call(
        body,
        out_shape=jax.ShapeDtypeStruct((m, n), jnp.float32),
        in_specs=[
            pl.BlockSpec(memory_space=pltpu.VMEM),
            pl.BlockSpec(memory_space=pltpu.VMEM),
        ],
        out_specs=pl.BlockSpec(memory_space=pltpu.MemorySpace.HBM),
        scratch_shapes=[
            pltpu.VMEM((2, mc, n), jnp.float32),
            pltpu.VMEM((2, mc, n), jnp.int8),
            pltpu.VMEM((4, mc, n), jnp.int8),
            pltpu.VMEM((2, 8, 128), jnp.float32),
            pltpu.VMEM((4, 8, 128), jnp.float32),
            pltpu.SemaphoreType.DMA((2,)),
            pltpu.SemaphoreType.DMA((2,)),
            pltpu.SemaphoreType.DMA((4,)),
            pltpu.SemaphoreType.DMA((2,)),
            pltpu.SemaphoreType.DMA((4,)),
        ],
        compiler_params=pltpu.CompilerParams(
            collective_id=0, vmem_limit_bytes=62 * 1024 * 1024
        ),
    )(A, B)
